# baseline (device time: 12871 ns/iter reference)
import jax
import jax.numpy as jnp
from jax import lax
from jax.experimental import pallas as pl
from jax.experimental.pallas import tpu as pltpu

N_DEV = 4
B, Sq, Skv, Hq, Dh = 2, 128, 512, 4, 64
D_MODEL = 512
S_PER = Skv // N_DEV
WINDOW = 128
SCALE = 0.125
HD = Hq * Dh
PW = HD + 2 * Hq
QH = Sq // 2
NCHUNK = 2 * B


def kernel(x, Wq, K_ext, V_ext, Wo):
    def body(x_ref, wq_ref, k_ref, v_ref, wo_ref, out_ref,
             recv0, recv1, send_sems, recv_sems):
        my = lax.axis_index("i")

        barrier_sem = pltpu.get_barrier_semaphore()
        for k in range(1, N_DEV):
            pl.semaphore_signal(
                barrier_sem, inc=1,
                device_id=((my + k) % N_DEV,),
                device_id_type=pl.DeviceIdType.MESH,
            )

        def compute_chunk(dst, q_b, b, qh, masked):
            lo = qh * QH
            us, ms, ls = [], [], []
            for h in range(Hq):
                q_c = q_b[lo:lo + QH, h * Dh:(h + 1) * Dh]
                k_bh = k_ref[b, :, h, :]
                v_bh = v_ref[b, :, h, :]
                s = lax.dot_general(
                    q_c, k_bh, (((1,), (1,)), ((), ())),
                    preferred_element_type=jnp.float32,
                ) * SCALE
                if masked:
                    qi = lax.broadcasted_iota(jnp.int32, (QH, S_PER), 0) + lo
                    kj = lax.broadcasted_iota(jnp.int32, (QH, S_PER), 1)
                    s = jnp.where(kj <= qi, s, -1e9)
                m = jnp.max(s, axis=1, keepdims=True)
                e = jnp.exp(s - m)
                l = jnp.sum(e, axis=1, keepdims=True)
                u = jnp.dot(e, v_bh, preferred_element_type=jnp.float32)
                us.append(u)
                ms.append(m)
                ls.append(l)
            dst[b, lo:lo + QH, 0:HD] = (
                jnp.concatenate(us, axis=1).astype(jnp.bfloat16))
            dst[b, lo:lo + QH, HD:PW] = (
                jnp.concatenate(ms + ls, axis=1).astype(jnp.bfloat16))

        def send_chunk(src, b, qh, targets, src_idx):
            c = b * 2 + qh
            for i, tgt in enumerate(targets):
                pltpu.make_async_remote_copy(
                    src_ref=src.at[b, pl.ds(qh * QH, QH)],
                    dst_ref=src.at[b, pl.ds(qh * QH, QH)],
                    send_sem=send_sems.at[c * 3 + i],
                    recv_sem=recv_sems.at[src_idx * NCHUNK + c],
                    device_id=(tgt,), device_id_type=pl.DeviceIdType.MESH,
                ).start()

        def sender(dst, targets, src_idx, masked):
            first = True
            for b in range(B):
                q_b = jnp.dot(x_ref[b], wq_ref[...],
                              preferred_element_type=jnp.float32)
                for qh in range(2):
                    compute_chunk(dst, q_b, b, qh, masked)
                    if first:
                        pl.semaphore_wait(barrier_sem, N_DEV - 1)
                        first = False
                    send_chunk(dst, b, qh, targets, src_idx)

        @pl.when(my == 0)
        def _():
            sender(recv0, [2, 1, 3], 0, masked=False)

        @pl.when(my == 1)
        def _():
            sender(recv1, [3, 2, 0], 1, masked=True)

        @pl.when(my >= 2)
        def _():
            pl.semaphore_wait(barrier_sem, N_DEV - 1)

        for b in range(B):
            for qh in range(2):
                c = b * 2 + qh
                lo = qh * QH

                @pl.when(my != 0)
                def _():
                    pltpu.make_async_remote_copy(
                        src_ref=recv0.at[b, pl.ds(lo, QH)],
                        dst_ref=recv0.at[b, pl.ds(lo, QH)],
                        send_sem=send_sems.at[0],
                        recv_sem=recv_sems.at[0 * NCHUNK + c],
                        device_id=(0,), device_id_type=pl.DeviceIdType.MESH,
                    ).wait_recv()

                @pl.when(my != 1)
                def _():
                    pltpu.make_async_remote_copy(
                        src_ref=recv1.at[b, pl.ds(lo, QH)],
                        dst_ref=recv1.at[b, pl.ds(lo, QH)],
                        send_sem=send_sems.at[0],
                        recv_sem=recv_sems.at[1 * NCHUNK + c],
                        device_id=(1,), device_id_type=pl.DeviceIdType.MESH,
                    ).wait_recv()

                ctxs = []
                for h in range(Hq):
                    u0 = recv0[b, lo:lo + QH, h * Dh:(h + 1) * Dh].astype(jnp.float32)
                    u1 = recv1[b, lo:lo + QH, h * Dh:(h + 1) * Dh].astype(jnp.float32)
                    m0 = recv0[b, lo:lo + QH, HD + h:HD + h + 1].astype(jnp.float32)
                    m1 = recv1[b, lo:lo + QH, HD + h:HD + h + 1].astype(jnp.float32)
                    l0 = recv0[b, lo:lo + QH, HD + Hq + h:HD + Hq + h + 1].astype(jnp.float32)
                    l1 = recv1[b, lo:lo + QH, HD + Hq + h:HD + Hq + h + 1].astype(jnp.float32)
                    m = jnp.maximum(m0, m1)
                    a0 = jnp.exp(m0 - m)
                    a1 = jnp.exp(m1 - m)
                    den = a0 * l0 + a1 * l1
                    ctxs.append((a0 * u0 + a1 * u1) / den)
                ctx_c = jnp.concatenate(ctxs, axis=1)
                out_ref[b, lo:lo + QH, :] = jnp.dot(
                    ctx_c, wo_ref[...], preferred_element_type=jnp.float32)

        @pl.when(my < 2)
        def _():
            for i in range(NCHUNK * 3):
                pltpu.make_async_remote_copy(
                    src_ref=recv0.at[0, pl.ds(0, QH)],
                    dst_ref=recv0.at[0, pl.ds(0, QH)],
                    send_sem=send_sems.at[i], recv_sem=recv_sems.at[0],
                    device_id=(0,), device_id_type=pl.DeviceIdType.MESH,
                ).wait_send()

    return pl.pallas_call(
        body,
        out_shape=jax.ShapeDtypeStruct((B, Sq, D_MODEL), jnp.float32),
        in_specs=[pl.BlockSpec(memory_space=pltpu.VMEM)] * 5,
        out_specs=pl.BlockSpec(memory_space=pltpu.VMEM),
        scratch_shapes=[
            pltpu.VMEM((B, Sq, PW), jnp.bfloat16),
            pltpu.VMEM((B, Sq, PW), jnp.bfloat16),
            pltpu.SemaphoreType.DMA((NCHUNK * 3,)),
            pltpu.SemaphoreType.DMA((2 * NCHUNK,)),
        ],
        compiler_params=pltpu.CompilerParams(collective_id=0),
    )(x, Wq, K_ext, V_ext, Wo)


# device time: 11835 ns/iter; 1.0875x vs baseline; 1.0875x over previous
import jax
import jax.numpy as jnp
from jax import lax
from jax.experimental import pallas as pl
from jax.experimental.pallas import tpu as pltpu

N_DEV = 4
B, Sq, Skv, Hq, Dh = 2, 128, 512, 4, 64
D_MODEL = 512
S_PER = Skv // N_DEV
WINDOW = 128
SCALE = 0.125
HD = Hq * Dh
PW = HD + 2 * Hq


def kernel(x, Wq, K_ext, V_ext, Wo):
    def body(x_ref, wq_ref, k_ref, v_ref, wo_ref, out_ref,
             recv0, recv1, send_sems, recv_sems):
        my = lax.axis_index("i")

        barrier_sem = pltpu.get_barrier_semaphore()
        for k in range(1, N_DEV):
            pl.semaphore_signal(
                barrier_sem, inc=1,
                device_id=((my + k) % N_DEV,),
                device_id_type=pl.DeviceIdType.MESH,
            )

        def compute_partial_b(dst, b, masked):
            q_b = jnp.dot(x_ref[b], wq_ref[...],
                          preferred_element_type=jnp.float32) * SCALE
            us, ms, ls = [], [], []
            for h in range(Hq):
                q_bh = q_b[:, h * Dh:(h + 1) * Dh]
                k_bh = k_ref[b, :, h, :]
                v_bh = v_ref[b, :, h, :]
                s = lax.dot_general(
                    q_bh, k_bh, (((1,), (1,)), ((), ())),
                    preferred_element_type=jnp.float32,
                )
                if masked:
                    qi = lax.broadcasted_iota(jnp.int32, (Sq, S_PER), 0)
                    kj = lax.broadcasted_iota(jnp.int32, (Sq, S_PER), 1)
                    s = jnp.where(kj <= qi, s, -1e9)
                m = jnp.max(s, axis=1, keepdims=True)
                e = jnp.exp(s - m)
                l = jnp.sum(e, axis=1, keepdims=True)
                u = jnp.dot(e, v_bh, preferred_element_type=jnp.float32)
                us.append(u)
                ms.append(m)
                ls.append(l)
            dst[b, :, 0:HD] = jnp.concatenate(us, axis=1).astype(jnp.bfloat16)
            dst[b, :, HD:PW] = jnp.concatenate(ms + ls, axis=1).astype(jnp.bfloat16)

        def send_chunk(src, b, targets, recv_sem_idx):
            for i, tgt in enumerate(targets):
                pltpu.make_async_remote_copy(
                    src_ref=src.at[b], dst_ref=src.at[b],
                    send_sem=send_sems.at[b * 3 + i],
                    recv_sem=recv_sems.at[recv_sem_idx * 2 + b],
                    device_id=(tgt,), device_id_type=pl.DeviceIdType.MESH,
                ).start()

        @pl.when(my == 0)
        def _():
            compute_partial_b(recv0, 0, masked=False)

        @pl.when(my == 1)
        def _():
            compute_partial_b(recv1, 0, masked=True)

        pl.semaphore_wait(barrier_sem, N_DEV - 1)

        @pl.when(my == 0)
        def _():
            send_chunk(recv0, 0, [2, 1, 3], 0)
            compute_partial_b(recv0, 1, masked=False)
            send_chunk(recv0, 1, [2, 1, 3], 0)

        @pl.when(my == 1)
        def _():
            send_chunk(recv1, 0, [3, 2, 0], 1)
            compute_partial_b(recv1, 1, masked=True)
            send_chunk(recv1, 1, [3, 2, 0], 1)

        for b in range(B):
            @pl.when(my != 0)
            def _():
                pltpu.make_async_remote_copy(
                    src_ref=recv0.at[b], dst_ref=recv0.at[b],
                    send_sem=send_sems.at[0], recv_sem=recv_sems.at[0 * 2 + b],
                    device_id=(0,), device_id_type=pl.DeviceIdType.MESH,
                ).wait_recv()

            @pl.when(my != 1)
            def _():
                pltpu.make_async_remote_copy(
                    src_ref=recv1.at[b], dst_ref=recv1.at[b],
                    send_sem=send_sems.at[0], recv_sem=recv_sems.at[1 * 2 + b],
                    device_id=(1,), device_id_type=pl.DeviceIdType.MESH,
                ).wait_recv()

            ctxs = []
            for h in range(Hq):
                u0 = recv0[b, :, h * Dh:(h + 1) * Dh].astype(jnp.float32)
                u1 = recv1[b, :, h * Dh:(h + 1) * Dh].astype(jnp.float32)
                m0 = recv0[b, :, HD + h:HD + h + 1].astype(jnp.float32)
                m1 = recv1[b, :, HD + h:HD + h + 1].astype(jnp.float32)
                l0 = recv0[b, :, HD + Hq + h:HD + Hq + h + 1].astype(jnp.float32)
                l1 = recv1[b, :, HD + Hq + h:HD + Hq + h + 1].astype(jnp.float32)
                m = jnp.maximum(m0, m1)
                a0 = jnp.exp(m0 - m)
                a1 = jnp.exp(m1 - m)
                den = a0 * l0 + a1 * l1
                ctxs.append((a0 * u0 + a1 * u1) / den)
            ctx_b = jnp.concatenate(ctxs, axis=1)
            out_ref[b] = jnp.dot(ctx_b, wo_ref[...],
                                 preferred_element_type=jnp.float32)

        @pl.when(my < 2)
        def _():
            for i in range(2 * 3):
                pltpu.make_async_remote_copy(
                    src_ref=recv0.at[0], dst_ref=recv0.at[0],
                    send_sem=send_sems.at[i], recv_sem=recv_sems.at[0],
                    device_id=(0,), device_id_type=pl.DeviceIdType.MESH,
                ).wait_send()

    return pl.pallas_call(
        body,
        out_shape=jax.ShapeDtypeStruct((B, Sq, D_MODEL), jnp.float32),
        in_specs=[pl.BlockSpec(memory_space=pltpu.VMEM)] * 5,
        out_specs=pl.BlockSpec(memory_space=pltpu.VMEM),
        scratch_shapes=[
            pltpu.VMEM((B, Sq, PW), jnp.bfloat16),
            pltpu.VMEM((B, Sq, PW), jnp.bfloat16),
            pltpu.SemaphoreType.DMA((2 * 3,)),
            pltpu.SemaphoreType.DMA((2 * 2,)),
        ],
        compiler_params=pltpu.CompilerParams(collective_id=0),
    )(x, Wq, K_ext, V_ext, Wo)
